# baseline (speedup 1.0000x reference)
import functools

import jax
import jax.numpy as jnp
from jax import lax
from jax.experimental import pallas as pl
from jax.experimental.pallas import tpu as pltpu

B, R, N, D = 4, 4, 1024, 256
NTILE = 256


def _layer_body(adj_ref, x_ref, xown_ref, wr_ref, br_ref, w0_ref, b0_ref,
                out_ref, agg_ref, den_ref):
    r = pl.program_id(2)

    adj_blk = adj_ref[0, 0]
    x_full = x_ref[0]

    s = jnp.dot(adj_blk, x_full, preferred_element_type=jnp.float32)
    rowsum = jnp.sum(adj_blk, axis=1, keepdims=True)

    wr = wr_ref[r]
    contrib = lax.dot_general(s, wr, (((1,), (1,)), ((), ())),
                              preferred_element_type=jnp.float32)
    contrib = contrib + rowsum * br_ref[r][None, :]

    @pl.when(r == 0)
    def _init():
        agg_ref[...] = contrib
        den_ref[...] = rowsum

    @pl.when(r > 0)
    def _acc():
        agg_ref[...] += contrib
        den_ref[...] += rowsum

    @pl.when(r == R - 1)
    def _finish():
        x_own = xown_ref[0]
        h0 = lax.dot_general(x_own, w0_ref[...], (((1,), (1,)), ((), ())),
                             preferred_element_type=jnp.float32)
        h0 = h0 + b0_ref[...]
        denoms = den_ref[...] + 1.0
        out_ref[0] = jnp.maximum((agg_ref[...] + h0) / denoms, 0.0)


def _layer(x, adj, w0l, b0l, wrl, brl):
    grid = (B, N // NTILE, R)
    return pl.pallas_call(
        _layer_body,
        grid=grid,
        in_specs=[
            pl.BlockSpec((1, 1, NTILE, N), lambda b, n, r: (b, r, n, 0)),
            pl.BlockSpec((1, N, D), lambda b, n, r: (b, 0, 0)),
            pl.BlockSpec((1, NTILE, D), lambda b, n, r: (b, n, 0)),
            pl.BlockSpec((R, D, D), lambda b, n, r: (0, 0, 0)),
            pl.BlockSpec((R, D), lambda b, n, r: (0, 0)),
            pl.BlockSpec((D, D), lambda b, n, r: (0, 0)),
            pl.BlockSpec((1, D), lambda b, n, r: (0, 0)),
        ],
        out_specs=pl.BlockSpec((1, NTILE, D), lambda b, n, r: (b, n, 0)),
        out_shape=jax.ShapeDtypeStruct((B, N, D), jnp.float32),
        scratch_shapes=[
            pltpu.VMEM((NTILE, D), jnp.float32),
            pltpu.VMEM((NTILE, 1), jnp.float32),
        ],
    )(adj, x, x, wrl, brl, w0l, b0l)


@jax.jit
def kernel(nodes, adj, W0, b0, Wr, br):
    x = nodes
    outs = []
    for l in range(W0.shape[0]):
        x = _layer(x, adj, W0[l], b0[l][None, :], Wr[l], br[l])
        outs.append(x)
    return tuple(outs)
